# Initial kernel scaffold; baseline (speedup 1.0000x reference)
import jax
import jax.numpy as jnp
from jax.experimental import pallas as pl


def kernel(inputs, table):
    raise NotImplementedError("write your pallas kernel here")


# speedup vs baseline: 1.4684x; 1.4684x over previous
import functools
import math

import jax
import jax.numpy as jnp
from jax import lax
from jax.experimental import pallas as pl
from jax.experimental.pallas import tpu as pltpu
from jax.experimental.pallas import tpu_sc as plsc

D_MODEL = 768
LANES = 16
NUM_CORES = 2
NUM_SUBCORES = 16
NUM_WORKERS = NUM_CORES * NUM_SUBCORES
SCALE = math.sqrt(D_MODEL)


def _make_sc_gather(n_rows: int):
    rows_per_worker = n_rows // NUM_WORKERS
    chunk = 64
    n_chunks = rows_per_worker // chunk
    vecs_per_row = D_MODEL // LANES

    mesh = plsc.VectorSubcoreMesh(core_axis_name="c", subcore_axis_name="s")

    @functools.partial(
        pl.kernel,
        out_type=jax.ShapeDtypeStruct((n_rows, D_MODEL), jnp.float32),
        mesh=mesh,
        scratch_types=[
            pltpu.VMEM((rows_per_worker,), jnp.int32),
            pltpu.VMEM((2, chunk, D_MODEL), jnp.float32),
            pltpu.SemaphoreType.DMA,
            pltpu.SemaphoreType.DMA,
        ],
    )
    def sc_gather(table_hbm, idx_hbm, out_hbm, idx_v, buf, sem0, sem1):
        wid = lax.axis_index("s") * NUM_CORES + lax.axis_index("c")
        base = wid * rows_per_worker
        pltpu.sync_copy(idx_hbm.at[pl.ds(base, rows_per_worker)], idx_v)

        sems = (sem0, sem1)
        copies = [None, None]
        copies[0] = pltpu.async_copy(
            table_hbm.at[idx_v.at[pl.ds(0, chunk)]], buf.at[0], sems[0]
        )
        for ch in range(n_chunks):
            cur = ch % 2
            copies[cur].wait()
            if ch + 1 < n_chunks:
                nxt = (ch + 1) % 2
                copies[nxt] = pltpu.async_copy(
                    table_hbm.at[idx_v.at[pl.ds((ch + 1) * chunk, chunk)]],
                    buf.at[nxt],
                    sems[nxt],
                )

            def scale_row(r, _, cur=cur):
                for k in range(vecs_per_row):
                    sl = pl.ds(k * LANES, LANES)
                    buf[cur, r, sl] = buf[cur, r, sl] * SCALE
                return _

            lax.fori_loop(0, chunk, scale_row, None)
            pltpu.sync_copy(
                buf.at[cur], out_hbm.at[pl.ds(base + ch * chunk, chunk)]
            )

    return sc_gather


def kernel(inputs, table):
    batch, seq_len = inputs.shape
    n_rows = batch * seq_len
    idx_flat = inputs.reshape(n_rows)
    gathered = _make_sc_gather(n_rows)(table, idx_flat)
    return gathered.reshape(batch, seq_len, D_MODEL)
